# Initial kernel scaffold; baseline (speedup 1.0000x reference)
import jax
import jax.numpy as jnp
from jax.experimental import pallas as pl


def kernel(seq_len_q, seq_len_k, embeddings_table):
    raise NotImplementedError("write your pallas kernel here")


# speedup vs baseline: 8.1477x; 8.1477x over previous
import functools

import jax
import jax.numpy as jnp
from jax import lax
from jax.experimental import pallas as pl
from jax.experimental.pallas import tpu as pltpu
from jax.experimental.pallas import tpu_sc as plsc

EMB = 32
SEQ = 2048
NPOS = 1025
HEAD = SEQ - 1 - (NPOS - 1) // 2
P_WORDS = (2 * HEAD + NPOS) * EMB
ROW_WORDS = SEQ * EMB
NW = 32
ROWS_PER_W = SEQ // NW
TAIL_OFF = (HEAD + NPOS) * EMB


def _sc_call(table1d):
    mesh = plsc.VectorSubcoreMesh(core_axis_name="c", subcore_axis_name="s")

    @functools.partial(
        pl.kernel,
        mesh=mesh,
        out_type=jax.ShapeDtypeStruct((SEQ * ROW_WORDS,), jnp.float32),
        scratch_types=[
            pltpu.VMEM((P_WORDS,), jnp.float32),
            pltpu.SemaphoreType.DMA,
        ],
    )
    def body(table_hbm, out_hbm, p_ref, sem):
        w = lax.axis_index("s") * 2 + lax.axis_index("c")

        pltpu.sync_copy(table_hbm, p_ref.at[pl.ds(HEAD * EMB, NPOS * EMB)])

        v0 = p_ref[pl.ds(HEAD * EMB, 16)]
        v1 = p_ref[pl.ds(HEAD * EMB + 16, 16)]
        v2 = p_ref[pl.ds((HEAD + NPOS - 1) * EMB, 16)]
        v3 = p_ref[pl.ds((HEAD + NPOS - 1) * EMB + 16, 16)]

        def fill(r, carry):
            base = r * EMB
            p_ref[pl.ds(base, 16)] = v0
            p_ref[pl.ds(base + 16, 16)] = v1
            p_ref[pl.ds(TAIL_OFF + base, 16)] = v2
            p_ref[pl.ds(TAIL_OFF + base + 16, 16)] = v3
            return carry

        lax.fori_loop(0, HEAD, fill, 0)

        first = w * ROWS_PER_W
        copies = []
        for t in range(ROWS_PER_W):
            i = first + t
            off = pl.multiple_of((SEQ - 1 - i) * EMB, EMB)
            dst = pl.multiple_of(i * ROW_WORDS, ROW_WORDS)
            copies.append(
                pltpu.async_copy(
                    p_ref.at[pl.ds(off, ROW_WORDS)],
                    out_hbm.at[pl.ds(dst, ROW_WORDS)],
                    sem,
                )
            )
        for cp in copies:
            cp.wait()

    return body(table1d)


def kernel(seq_len_q, seq_len_k, embeddings_table):
    del seq_len_q, seq_len_k
    out = _sc_call(embeddings_table.reshape(-1))
    return out.reshape(SEQ, SEQ, EMB)


# speedup vs baseline: 8.1484x; 1.0001x over previous
import functools

import jax
import jax.numpy as jnp
from jax import lax
from jax.experimental import pallas as pl
from jax.experimental.pallas import tpu as pltpu
from jax.experimental.pallas import tpu_sc as plsc

EMB = 32
SEQ = 2048
NPOS = 1025
HEAD = SEQ - 1 - (NPOS - 1) // 2
P_WORDS = (2 * HEAD + NPOS) * EMB
ROW_WORDS = SEQ * EMB
NW = 32
ROWS_PER_W = SEQ // NW
TAIL_OFF = (HEAD + NPOS) * EMB


def _sc_call(table1d):
    mesh = plsc.VectorSubcoreMesh(core_axis_name="c", subcore_axis_name="s")

    @functools.partial(
        pl.kernel,
        mesh=mesh,
        out_type=jax.ShapeDtypeStruct((SEQ, SEQ, EMB), jnp.float32),
        scratch_types=[
            pltpu.VMEM((2 * HEAD + NPOS, EMB), jnp.float32),
            pltpu.SemaphoreType.DMA,
        ],
        compiler_params=pltpu.CompilerParams(use_tc_tiling_on_sc=False),
    )
    def body(table_hbm, out_hbm, p_ref, sem):
        w = lax.axis_index("s") * 2 + lax.axis_index("c")

        pltpu.sync_copy(table_hbm, p_ref.at[pl.ds(HEAD, NPOS), :])

        v0 = p_ref[HEAD, 0:16]
        v1 = p_ref[HEAD, 16:32]
        v2 = p_ref[HEAD + NPOS - 1, 0:16]
        v3 = p_ref[HEAD + NPOS - 1, 16:32]

        def fill(r, carry):
            p_ref[r, 0:16] = v0
            p_ref[r, 16:32] = v1
            p_ref[HEAD + NPOS + r, 0:16] = v2
            p_ref[HEAD + NPOS + r, 16:32] = v3
            return carry

        lax.fori_loop(0, HEAD, fill, 0)

        first = w * ROWS_PER_W
        copies = []
        for t in range(ROWS_PER_W):
            i = first + t
            off = pl.multiple_of(SEQ - 1 - i, 1)
            copies.append(
                pltpu.async_copy(
                    p_ref.at[pl.ds(off, SEQ), :],
                    out_hbm.at[i],
                    sem,
                )
            )
        for cp in copies:
            cp.wait()

    return body(table1d)


def kernel(seq_len_q, seq_len_k, embeddings_table):
    del seq_len_q, seq_len_k
    return _sc_call(embeddings_table)


# speedup vs baseline: 79.8342x; 9.7975x over previous
import functools

import jax
import jax.numpy as jnp
from jax import lax
from jax.experimental import pallas as pl
from jax.experimental.pallas import tpu as pltpu
from jax.experimental.pallas import tpu_sc as plsc

EMB = 32
SEQ = 2048
NPOS = 1025
HEAD = 1535
NW = 32
ROWS_PER_W = SEQ // NW
WCOLS = SEQ + ROWS_PER_W
ROW_WORDS = SEQ * EMB
CHUNK = 8 * SEQ
TT_W = EMB * NPOS
WT_OFF = TT_W
ARENA = TT_W + EMB * WCOLS


def _sc_call(table_t):
    mesh = plsc.VectorSubcoreMesh(core_axis_name="c", subcore_axis_name="s")

    @functools.partial(
        pl.kernel,
        mesh=mesh,
        out_type=jax.ShapeDtypeStruct((SEQ * ROW_WORDS,), jnp.float32),
        scratch_types=[
            pltpu.VMEM((ARENA,), jnp.float32),
            pltpu.SemaphoreType.DMA,
            pltpu.SemaphoreType.DMA,
        ],
        compiler_params=pltpu.CompilerParams(needs_layout_passes=False),
    )
    def body(tt_hbm, out_hbm, arena, sem0, sem1):
        w = lax.axis_index("s") * 2 + lax.axis_index("c")
        sems = (sem0, sem1)

        pltpu.sync_copy(tt_hbm, arena.at[pl.ds(0, TT_W)])

        u0 = 1984 - ROWS_PER_W * w
        lanes = lax.iota(jnp.int32, 16)

        def build(k, carry):
            c = k // (WCOLS // 16)
            xv = k % (WCOLS // 16)
            lo = c * NPOS
            base = lo + u0 + 16 * xv - HEAD
            idx = jnp.clip(lanes + base, lo, lo + NPOS - 1)
            vals = plsc.load_gather(arena, [idx])
            arena[pl.ds(WT_OFF + c * WCOLS + 16 * xv, 16)] = vals
            return carry

        lax.fori_loop(0, EMB * (WCOLS // 16), build, 0)

        first = w * ROWS_PER_W

        def emit_row(t, carry):
            delta = (ROWS_PER_W - 1) - t
            for tr in range(4):
                b = tr % 2

                def reclaim(b=b):
                    pltpu.make_async_copy(
                        arena.at[pl.ds(b * CHUNK, CHUNK)],
                        out_hbm.at[pl.ds(0, CHUNK)],
                        sems[b],
                    ).wait()

                if tr < 2:
                    pl.when(t > 0)(reclaim)
                else:
                    reclaim()

                @plsc.parallel_loop(0, 128, step=1, unroll=8)
                def pack(blk):
                    tc = blk // 8
                    r = blk % 8
                    src = WT_OFF + (8 * tr + r) * WCOLS + delta + 128 * tc
                    dst = b * CHUNK + blk * 128
                    for q in range(8):
                        arena[pl.ds(dst + 16 * q, 16)] = arena[
                            pl.ds(src + 16 * q, 16)
                        ]

                out_off = pl.multiple_of(
                    (first + t) * ROW_WORDS + tr * CHUNK, CHUNK
                )
                pltpu.async_copy(
                    arena.at[pl.ds(b * CHUNK, CHUNK)],
                    out_hbm.at[pl.ds(out_off, CHUNK)],
                    sems[b],
                )
            return carry

        lax.fori_loop(0, ROWS_PER_W, emit_row, 0)

        for b in range(2):
            pltpu.make_async_copy(
                arena.at[pl.ds(b * CHUNK, CHUNK)],
                out_hbm.at[pl.ds(0, CHUNK)],
                sems[b],
            ).wait()

    return body(table_t)


def kernel(seq_len_q, seq_len_k, embeddings_table):
    del seq_len_q, seq_len_k
    flat = _sc_call(embeddings_table.T.reshape(-1))
    s = flat.reshape(SEQ, 4, 16, 8, 128)
    return s.transpose(0, 2, 4, 1, 3).reshape(SEQ, SEQ, EMB)
